# baseline (device time: 211483 ns/iter reference)
import jax
import jax.numpy as jnp
from jax import lax
from jax.experimental import pallas as pl
from jax.experimental.pallas import tpu as pltpu

N_DEV = 4
SQ = 256
D_MODEL = 1024
SKV = 4096
H_TOT = 32
H_LOC = 8
DH = 128
SCALE = 0.08838834764831843
WIN = 512
G = 32
NEG = -1e9
F32 = jnp.float32


def _dot_nt(a, b):
    return lax.dot_general(a, b, (((1,), (1,)), ((), ())),
                           preferred_element_type=F32)


def _dot(a, b):
    return jnp.dot(a, b, preferred_element_type=F32)


def kernel(x, Wq, K_ext, V_ext, Wo):
    x2 = x.reshape(SQ, D_MODEL)
    k2 = K_ext.reshape(SKV, H_TOT * DH)
    v2 = V_ext.reshape(SKV, H_TOT * DH)

    def body(x_ref, wq_ref, k_hbm, v_hbm, wo_ref, out_ref,
             comm, q_ref, kbuf, vbuf, partial, rs_recv, rs_send,
             ag_ssem, ag_rsem, k_sems, v_sems, rs_ssem, rs_rsem):
        me = lax.axis_index("i")
        right = (me + 1) % N_DEV
        left = (me + 3) % N_DEV
        h0col = me * (H_LOC * DH)

        kcp = {}
        vcp = {}

        def start_kv(h, slot):
            col = h0col + h * DH
            kcp[h] = pltpu.make_async_copy(
                k_hbm.at[:, pl.ds(col, DH)], kbuf.at[slot], k_sems.at[slot])
            vcp[h] = pltpu.make_async_copy(
                v_hbm.at[:, pl.ds(col, DH)], vbuf.at[slot], v_sems.at[slot])
            kcp[h].start()
            vcp[h].start()

        start_kv(0, 0)

        bsem = pltpu.get_barrier_semaphore()
        pl.semaphore_signal(bsem, inc=1, device_id=(left,),
                            device_id_type=pl.DeviceIdType.MESH)
        pl.semaphore_signal(bsem, inc=1, device_id=(right,),
                            device_id_type=pl.DeviceIdType.MESH)
        pl.semaphore_wait(bsem, 2)

        q_ref[pl.ds(me * SQ, SQ), :] = _dot(x_ref[...], wq_ref[...])
        for h in range(N_DEV - 1):
            src = x_ref if h == 0 else comm.at[(h - 1) % 2]
            rdma = pltpu.make_async_remote_copy(
                src_ref=src, dst_ref=comm.at[h % 2],
                send_sem=ag_ssem.at[h], recv_sem=ag_rsem.at[h],
                device_id=(right,), device_id_type=pl.DeviceIdType.MESH)
            rdma.start()
            rdma.wait()
            origin = (me + 3 - h) % N_DEV
            q_ref[pl.ds(origin * SQ, SQ), :] = _dot(comm[h % 2], wq_ref[...])

        for h in range(H_LOC):
            slot = h % 2
            kcp[h].wait()
            vcp[h].wait()
            if h + 1 < H_LOC:
                start_kv(h + 1, (h + 1) % 2)

            qg = q_ref[0:G, h * DH:(h + 1) * DH]
            sg = _dot_nt(qg, kbuf[slot]) * SCALE
            mg = jnp.max(sg, axis=1, keepdims=True)
            eg = jnp.exp(sg - mg)
            wg = eg / jnp.sum(eg, axis=1, keepdims=True)
            ctx_g = _dot(wg, vbuf[slot])

            wo_h = wo_ref[h * DH:(h + 1) * DH, :]
            for b in range(N_DEV):
                ws = 0 if b == 0 else b * SQ - 128
                qb = q_ref[b * SQ:(b + 1) * SQ, h * DH:(h + 1) * DH]
                kw = kbuf[slot, ws:ws + WIN, :]
                sw = _dot_nt(qb, kw) * SCALE
                qi = lax.broadcasted_iota(jnp.int32, (SQ, WIN), 0) + b * SQ
                ki = lax.broadcasted_iota(jnp.int32, (SQ, WIN), 1) + ws
                mask = jnp.abs(qi - ki) <= 128
                if b == 0:
                    mask = mask | (ki < G)
                sw = jnp.where(mask, sw, NEG)
                if b == 0:
                    m = jnp.max(sw, axis=1, keepdims=True)
                    ew = jnp.exp(sw - m)
                    den = jnp.sum(ew, axis=1, keepdims=True)
                    ctx = _dot(ew / den, vbuf[slot, ws:ws + WIN, :])
                    ctx = jnp.concatenate([ctx_g, ctx[G:]], axis=0)
                else:
                    kp = kbuf[slot, 0:DH, :]
                    sp = _dot_nt(qb, kp) * SCALE
                    kip = lax.broadcasted_iota(jnp.int32, (SQ, DH), 1)
                    sp = jnp.where(kip < G, sp, NEG)
                    m = jnp.maximum(jnp.max(sw, axis=1, keepdims=True),
                                    jnp.max(sp, axis=1, keepdims=True))
                    ew = jnp.exp(sw - m)
                    ep = jnp.exp(sp - m)
                    den = (jnp.sum(ew, axis=1, keepdims=True)
                           + jnp.sum(ep, axis=1, keepdims=True))
                    ctx = (_dot(ew / den, vbuf[slot, ws:ws + WIN, :])
                           + _dot(ep / den, vbuf[slot, 0:DH, :]))
                pb = _dot(ctx, wo_h)
                if h == 0:
                    partial[b * SQ:(b + 1) * SQ, :] = pb
                else:
                    partial[b * SQ:(b + 1) * SQ, :] = (
                        partial[b * SQ:(b + 1) * SQ, :] + pb)

        for t in range(N_DEV - 1):
            blk = (me + 3 - t) % N_DEV
            acc = partial[pl.ds(blk * SQ, SQ), :]
            if t > 0:
                acc = acc + rs_recv[t - 1]
            rs_send[...] = acc
            rdma = pltpu.make_async_remote_copy(
                src_ref=rs_send, dst_ref=rs_recv.at[t],
                send_sem=rs_ssem.at[t], recv_sem=rs_rsem.at[t],
                device_id=(right,), device_id_type=pl.DeviceIdType.MESH)
            rdma.start()
            rdma.wait()
        out_ref[...] = rs_recv[N_DEV - 2] + partial[pl.ds(me * SQ, SQ), :]

    out = pl.pallas_call(
        body,
        out_shape=jax.ShapeDtypeStruct((SQ, D_MODEL), F32),
        in_specs=[
            pl.BlockSpec(memory_space=pltpu.MemorySpace.VMEM),
            pl.BlockSpec(memory_space=pltpu.MemorySpace.VMEM),
            pl.BlockSpec(memory_space=pl.ANY),
            pl.BlockSpec(memory_space=pl.ANY),
            pl.BlockSpec(memory_space=pltpu.MemorySpace.VMEM),
        ],
        out_specs=pl.BlockSpec(memory_space=pltpu.MemorySpace.VMEM),
        scratch_shapes=[
            pltpu.VMEM((2, SQ, D_MODEL), F32),
            pltpu.VMEM((N_DEV * SQ, D_MODEL), F32),
            pltpu.VMEM((2, SKV, DH), F32),
            pltpu.VMEM((2, SKV, DH), F32),
            pltpu.VMEM((N_DEV * SQ, D_MODEL), F32),
            pltpu.VMEM((N_DEV - 1, SQ, D_MODEL), F32),
            pltpu.VMEM((SQ, D_MODEL), F32),
            pltpu.SemaphoreType.DMA((N_DEV - 1,)),
            pltpu.SemaphoreType.DMA((N_DEV - 1,)),
            pltpu.SemaphoreType.DMA((2,)),
            pltpu.SemaphoreType.DMA((2,)),
            pltpu.SemaphoreType.DMA((N_DEV - 1,)),
            pltpu.SemaphoreType.DMA((N_DEV - 1,)),
        ],
        compiler_params=pltpu.CompilerParams(collective_id=0),
    )(x2, Wq, k2, v2, Wo)
    return out.reshape(1, SQ, D_MODEL)


# device time: 96803 ns/iter; 2.1847x vs baseline; 2.1847x over previous
import jax
import jax.numpy as jnp
from jax import lax
from jax.experimental import pallas as pl
from jax.experimental.pallas import tpu as pltpu

N_DEV = 4
SQ = 256
D_MODEL = 1024
SKV = 4096
H_TOT = 32
H_LOC = 8
DH = 128
SCALE = 0.08838834764831843
WIN = 512
G = 32
NEG = -1e9
F32 = jnp.float32


def _dot_nt(a, b):
    return lax.dot_general(a, b, (((1,), (1,)), ((), ())),
                           preferred_element_type=F32)


def _dot(a, b):
    return jnp.dot(a, b, preferred_element_type=F32)


def kernel(x, Wq, K_ext, V_ext, Wo):
    x2 = x.reshape(SQ, D_MODEL)

    def body(x_ref, wq_ref, k_hbm, v_hbm, wo_ref, out_ref,
             comm, q_ref, kbuf, vbuf, partial, rs_recv, rs_send,
             ag_ssem, ag_rsem, k_sems, v_sems, rs_ssem, rs_rsem):
        me = lax.axis_index("i")
        right = (me + 1) % N_DEV
        left = (me + 3) % N_DEV
        h0 = me * H_LOC

        kv_cps = []
        for h in range(H_LOC):
            kc = pltpu.make_async_copy(
                k_hbm.at[0, :, h0 + h, :], kbuf.at[h], k_sems.at[h])
            vc = pltpu.make_async_copy(
                v_hbm.at[0, :, h0 + h, :], vbuf.at[h], v_sems.at[h])
            kc.start()
            vc.start()
            kv_cps += [kc, vc]

        bsem = pltpu.get_barrier_semaphore()
        pl.semaphore_signal(bsem, inc=1, device_id=(left,),
                            device_id_type=pl.DeviceIdType.MESH)
        pl.semaphore_signal(bsem, inc=1, device_id=(right,),
                            device_id_type=pl.DeviceIdType.MESH)
        pl.semaphore_wait(bsem, 2)

        def ag_rdma(hop):
            src = x_ref if hop == 0 else comm.at[hop - 1]
            return pltpu.make_async_remote_copy(
                src_ref=src, dst_ref=comm.at[hop],
                send_sem=ag_ssem.at[hop], recv_sem=ag_rsem.at[hop],
                device_id=(right,), device_id_type=pl.DeviceIdType.MESH)

        def rs_rdma(t):
            return pltpu.make_async_remote_copy(
                src_ref=rs_send.at[t], dst_ref=rs_recv.at[t],
                send_sem=rs_ssem.at[t], recv_sem=rs_rsem.at[t],
                device_id=(right,), device_id_type=pl.DeviceIdType.MESH)

        ag = [ag_rdma(hop) for hop in range(N_DEV - 1)]
        rs = [rs_rdma(t) for t in range(N_DEV - 1)]

        def attn_block(b):
            ws = jnp.maximum(b * SQ - 128, 0)
            q0 = b * SQ
            for h in range(H_LOC):
                qb = q_ref[pl.ds(q0, SQ), h * DH:(h + 1) * DH]
                kw = kbuf[h, pl.ds(ws, WIN), :]
                sw = _dot_nt(qb, kw) * SCALE
                qi = lax.broadcasted_iota(jnp.int32, (SQ, WIN), 0) + q0
                ki = lax.broadcasted_iota(jnp.int32, (SQ, WIN), 1) + ws
                mask = (jnp.abs(qi - ki) <= 128) | ((ki < G) & (ws == 0))
                sw = jnp.where(mask, sw, NEG)
                kp = kbuf[h, 0:DH, :]
                sp = _dot_nt(qb, kp) * SCALE
                kip = lax.broadcasted_iota(jnp.int32, (SQ, DH), 1)
                sp = jnp.where((kip < G) & (ws > 0), sp, NEG)
                m = jnp.maximum(jnp.max(sw, axis=1, keepdims=True),
                                jnp.max(sp, axis=1, keepdims=True))
                ew = jnp.exp(sw - m)
                ep = jnp.exp(sp - m)
                den = (jnp.sum(ew, axis=1, keepdims=True)
                       + jnp.sum(ep, axis=1, keepdims=True))
                ctx = (_dot(ew / den, vbuf[h, pl.ds(ws, WIN), :])
                       + _dot(ep / den, vbuf[h, 0:DH, :]))
                rowi = lax.broadcasted_iota(jnp.int32, (SQ, DH), 0) + q0
                ctx = jnp.where(rowi < G, 0.0, ctx)
                pb = _dot(ctx, wo_ref[h * DH:(h + 1) * DH, :])
                if h == 0:
                    partial[pl.ds(q0, SQ), :] = pb
                else:
                    partial[pl.ds(q0, SQ), :] = partial[pl.ds(q0, SQ), :] + pb

            @pl.when(b == 0)
            def _():
                for h in range(H_LOC):
                    qg = q_ref[0:G, h * DH:(h + 1) * DH]
                    sg = _dot_nt(qg, kbuf[h]) * SCALE
                    mg = jnp.max(sg, axis=1, keepdims=True)
                    eg = jnp.exp(sg - mg)
                    wg = eg / jnp.sum(eg, axis=1, keepdims=True)
                    ctx_g = _dot(wg, vbuf[h])
                    g = _dot(ctx_g, wo_ref[h * DH:(h + 1) * DH, :])
                    gsum = g if h == 0 else gsum + g
                partial[0:G, :] = partial[0:G, :] + gsum

        ag[0].start()
        q_ref[pl.ds(me * SQ, SQ), :] = _dot(x_ref[...], wq_ref[...])
        for cp in kv_cps:
            cp.wait()
        attn_block(me)

        for p in range(1, N_DEV):
            ag[p - 1].wait_recv()
            if p < N_DEV - 1:
                ag[p].start()
            b = (me + N_DEV - p) % N_DEV
            q_ref[pl.ds(b * SQ, SQ), :] = _dot(comm[p - 1], wq_ref[...])
            if p == 2:
                blk0 = (me + 3) % N_DEV
                rs_send[0] = partial[pl.ds(blk0 * SQ, SQ), :]
                rs[0].start()
            attn_block(b)
            if p == 2:
                rs[0].wait_recv()
                blk1 = (me + 2) % N_DEV
                rs_send[1] = rs_recv[0] + partial[pl.ds(blk1 * SQ, SQ), :]
                rs[1].start()
            if p == 3:
                rs[1].wait_recv()
                blk2 = (me + 1) % N_DEV
                rs_send[2] = rs_recv[1] + partial[pl.ds(blk2 * SQ, SQ), :]
                rs[2].start()

        rs[2].wait_recv()
        out_ref[...] = rs_recv[2] + partial[pl.ds(me * SQ, SQ), :]

        for r in ag + rs:
            r.wait_send()

    out = pl.pallas_call(
        body,
        out_shape=jax.ShapeDtypeStruct((SQ, D_MODEL), F32),
        in_specs=[
            pl.BlockSpec(memory_space=pltpu.MemorySpace.VMEM),
            pl.BlockSpec(memory_space=pltpu.MemorySpace.VMEM),
            pl.BlockSpec(memory_space=pl.ANY),
            pl.BlockSpec(memory_space=pl.ANY),
            pl.BlockSpec(memory_space=pltpu.MemorySpace.VMEM),
        ],
        out_specs=pl.BlockSpec(memory_space=pltpu.MemorySpace.VMEM),
        scratch_shapes=[
            pltpu.VMEM((N_DEV - 1, SQ, D_MODEL), F32),
            pltpu.VMEM((N_DEV * SQ, D_MODEL), F32),
            pltpu.VMEM((H_LOC, SKV, DH), F32),
            pltpu.VMEM((H_LOC, SKV, DH), F32),
            pltpu.VMEM((N_DEV * SQ, D_MODEL), F32),
            pltpu.VMEM((N_DEV - 1, SQ, D_MODEL), F32),
            pltpu.VMEM((N_DEV - 1, SQ, D_MODEL), F32),
            pltpu.SemaphoreType.DMA((N_DEV - 1,)),
            pltpu.SemaphoreType.DMA((N_DEV - 1,)),
            pltpu.SemaphoreType.DMA((H_LOC,)),
            pltpu.SemaphoreType.DMA((H_LOC,)),
            pltpu.SemaphoreType.DMA((N_DEV - 1,)),
            pltpu.SemaphoreType.DMA((N_DEV - 1,)),
        ],
        compiler_params=pltpu.CompilerParams(
            collective_id=0, vmem_limit_bytes=100 * 1024 * 1024),
    )(x2, Wq, K_ext, V_ext, Wo)
    return out.reshape(1, SQ, D_MODEL)


# device time: 71046 ns/iter; 2.9767x vs baseline; 1.3625x over previous
import jax
import jax.numpy as jnp
from jax import lax
from jax.experimental import pallas as pl
from jax.experimental.pallas import tpu as pltpu

N_DEV = 4
SQ = 256
D_MODEL = 1024
SKV = 4096
H_TOT = 32
H_LOC = 8
DH = 128
SCALE = 0.08838834764831843
WIN = 512
G = 32
NEG = -1e9
F32 = jnp.float32


def _dot_nt(a, b):
    return lax.dot_general(a, b, (((1,), (1,)), ((), ())),
                           preferred_element_type=F32)


def _dot(a, b):
    return jnp.dot(a, b, preferred_element_type=F32)


def kernel(x, Wq, K_ext, V_ext, Wo):
    x2 = x.reshape(SQ, D_MODEL)

    def body(x_ref, wq_ref, k_hbm, v_hbm, wo_ref, out_ref,
             xb, comm, q_ref, kbuf, vbuf, partial, rs_recv, rs_send,
             ag_ssem, ag_rsem, k_sems, v_sems, rs_ssem, rs_rsem):
        me = lax.axis_index("i")
        right = (me + 1) % N_DEV
        left = (me + 3) % N_DEV
        h0 = me * H_LOC

        kv_cps = []
        for h in range(H_LOC):
            kc = pltpu.make_async_copy(
                k_hbm.at[0, :, h0 + h, :], kbuf.at[h], k_sems.at[h])
            vc = pltpu.make_async_copy(
                v_hbm.at[0, :, h0 + h, :], vbuf.at[h], v_sems.at[h])
            kc.start()
            vc.start()
            kv_cps += [kc, vc]

        bsem = pltpu.get_barrier_semaphore()
        pl.semaphore_signal(bsem, inc=1, device_id=(left,),
                            device_id_type=pl.DeviceIdType.MESH)
        pl.semaphore_signal(bsem, inc=1, device_id=(right,),
                            device_id_type=pl.DeviceIdType.MESH)
        pl.semaphore_wait(bsem, 2)

        xb[...] = x_ref[...].astype(jnp.bfloat16)

        def ag_rdma(hop):
            src = xb if hop == 0 else comm.at[hop - 1]
            return pltpu.make_async_remote_copy(
                src_ref=src, dst_ref=comm.at[hop],
                send_sem=ag_ssem.at[hop], recv_sem=ag_rsem.at[hop],
                device_id=(right,), device_id_type=pl.DeviceIdType.MESH)

        def rs_rdma(t):
            return pltpu.make_async_remote_copy(
                src_ref=rs_send.at[t], dst_ref=rs_recv.at[t],
                send_sem=rs_ssem.at[t], recv_sem=rs_rsem.at[t],
                device_id=(right,), device_id_type=pl.DeviceIdType.MESH)

        ag = [ag_rdma(hop) for hop in range(N_DEV - 1)]
        rs = [rs_rdma(t) for t in range(N_DEV - 1)]

        def attn_block(b):
            ws = jnp.maximum(b * SQ - 128, 0)
            q0 = b * SQ
            for h in range(H_LOC):
                qb = q_ref[pl.ds(q0, SQ), h * DH:(h + 1) * DH]
                kw = kbuf[h, pl.ds(ws, WIN), :]
                sw = _dot_nt(qb, kw) * SCALE
                qi = lax.broadcasted_iota(jnp.int32, (SQ, WIN), 0) + q0
                ki = lax.broadcasted_iota(jnp.int32, (SQ, WIN), 1) + ws
                mask = (jnp.abs(qi - ki) <= 128) | ((ki < G) & (ws == 0))
                sw = jnp.where(mask, sw, NEG)
                kp = kbuf[h, 0:DH, :]
                sp = _dot_nt(qb, kp) * SCALE
                kip = lax.broadcasted_iota(jnp.int32, (SQ, DH), 1)
                sp = jnp.where((kip < G) & (ws > 0), sp, NEG)
                m = jnp.maximum(jnp.max(sw, axis=1, keepdims=True),
                                jnp.max(sp, axis=1, keepdims=True))
                ew = jnp.exp(sw - m)
                ep = jnp.exp(sp - m)
                den = (jnp.sum(ew, axis=1, keepdims=True)
                       + jnp.sum(ep, axis=1, keepdims=True))
                ctx = (_dot(ew / den, vbuf[h, pl.ds(ws, WIN), :])
                       + _dot(ep / den, vbuf[h, 0:DH, :]))
                rowi = lax.broadcasted_iota(jnp.int32, (SQ, DH), 0) + q0
                ctx = jnp.where(rowi < G, 0.0, ctx)
                pb = _dot(ctx, wo_ref[h * DH:(h + 1) * DH, :])
                if h == 0:
                    partial[pl.ds(q0, SQ), :] = pb
                else:
                    partial[pl.ds(q0, SQ), :] = partial[pl.ds(q0, SQ), :] + pb

            @pl.when(b == 0)
            def _():
                for h in range(H_LOC):
                    qg = q_ref[0:G, h * DH:(h + 1) * DH]
                    sg = _dot_nt(qg, kbuf[h]) * SCALE
                    mg = jnp.max(sg, axis=1, keepdims=True)
                    eg = jnp.exp(sg - mg)
                    wg = eg / jnp.sum(eg, axis=1, keepdims=True)
                    ctx_g = _dot(wg, vbuf[h])
                    g = _dot(ctx_g, wo_ref[h * DH:(h + 1) * DH, :])
                    gsum = g if h == 0 else gsum + g
                partial[0:G, :] = partial[0:G, :] + gsum

        ag[0].start()
        q_ref[pl.ds(me * SQ, SQ), :] = _dot(x_ref[...], wq_ref[...])
        for cp in kv_cps:
            cp.wait()
        attn_block(me)

        for p in range(1, N_DEV):
            ag[p - 1].wait_recv()
            if p < N_DEV - 1:
                ag[p].start()
            b = (me + N_DEV - p) % N_DEV
            q_ref[pl.ds(b * SQ, SQ), :] = _dot(comm[p - 1], wq_ref[...])
            if p == 2:
                blk0 = (me + 3) % N_DEV
                rs_send[0] = partial[pl.ds(blk0 * SQ, SQ), :].astype(jnp.bfloat16)
                rs[0].start()
            attn_block(b)
            if p == 2:
                rs[0].wait_recv()
                blk1 = (me + 2) % N_DEV
                rs_send[1] = (rs_recv[0] + partial[pl.ds(blk1 * SQ, SQ), :]
                              ).astype(jnp.bfloat16)
                rs[1].start()
            if p == 3:
                rs[1].wait_recv()
                blk2 = (me + 1) % N_DEV
                rs_send[2] = (rs_recv[1] + partial[pl.ds(blk2 * SQ, SQ), :]
                              ).astype(jnp.bfloat16)
                rs[2].start()

        rs[2].wait_recv()
        out_ref[...] = rs_recv[2] + partial[pl.ds(me * SQ, SQ), :]

        for r in ag + rs:
            r.wait_send()

    out = pl.pallas_call(
        body,
        out_shape=jax.ShapeDtypeStruct((SQ, D_MODEL), F32),
        in_specs=[
            pl.BlockSpec(memory_space=pltpu.MemorySpace.VMEM),
            pl.BlockSpec(memory_space=pltpu.MemorySpace.VMEM),
            pl.BlockSpec(memory_space=pl.ANY),
            pl.BlockSpec(memory_space=pl.ANY),
            pl.BlockSpec(memory_space=pltpu.MemorySpace.VMEM),
        ],
        out_specs=pl.BlockSpec(memory_space=pltpu.MemorySpace.VMEM),
        scratch_shapes=[
            pltpu.VMEM((SQ, D_MODEL), jnp.bfloat16),
            pltpu.VMEM((N_DEV - 1, SQ, D_MODEL), jnp.bfloat16),
            pltpu.VMEM((N_DEV * SQ, D_MODEL), F32),
            pltpu.VMEM((H_LOC, SKV, DH), F32),
            pltpu.VMEM((H_LOC, SKV, DH), F32),
            pltpu.VMEM((N_DEV * SQ, D_MODEL), F32),
            pltpu.VMEM((N_DEV - 1, SQ, D_MODEL), jnp.bfloat16),
            pltpu.VMEM((N_DEV - 1, SQ, D_MODEL), jnp.bfloat16),
            pltpu.SemaphoreType.DMA((N_DEV - 1,)),
            pltpu.SemaphoreType.DMA((N_DEV - 1,)),
            pltpu.SemaphoreType.DMA((H_LOC,)),
            pltpu.SemaphoreType.DMA((H_LOC,)),
            pltpu.SemaphoreType.DMA((N_DEV - 1,)),
            pltpu.SemaphoreType.DMA((N_DEV - 1,)),
        ],
        compiler_params=pltpu.CompilerParams(
            collective_id=0, vmem_limit_bytes=100 * 1024 * 1024),
    )(x2, Wq, K_ext, V_ext, Wo)
    return out.reshape(1, SQ, D_MODEL)


# device time: 66925 ns/iter; 3.1600x vs baseline; 1.0616x over previous
import jax
import jax.numpy as jnp
from jax import lax
from jax.experimental import pallas as pl
from jax.experimental.pallas import tpu as pltpu

N_DEV = 4
SQ = 256
D_MODEL = 1024
SKV = 4096
H_TOT = 32
H_LOC = 8
DH = 128
SCALE = 0.08838834764831843
WIN = 512
G = 32
NEG = -1e9
F32 = jnp.float32
BF16 = jnp.bfloat16


def _dot_nt(a, b):
    return lax.dot_general(a, b, (((1,), (1,)), ((), ())),
                           preferred_element_type=F32)


def _dot(a, b):
    return jnp.dot(a, b, preferred_element_type=F32)


def kernel(x, Wq, K_ext, V_ext, Wo):
    x2 = x.reshape(SQ, D_MODEL)

    def body(x_ref, wq_ref, k_hbm, v_hbm, wo_ref, out_ref,
             xb, comm, q_ref, kstage, vstage, kb, vb, wqb, wob, ctxbuf,
             partial, rs_recv, rs_send,
             ag_ssem, ag_rsem, k_sems, v_sems, rs_ssem, rs_rsem):
        me = lax.axis_index("i")
        right = (me + 1) % N_DEV
        left = (me + 3) % N_DEV
        h0 = me * H_LOC

        kv_cp = {}

        def start_kv(h):
            s = h % 2
            kc = pltpu.make_async_copy(
                k_hbm.at[0, :, h0 + h, :], kstage.at[s], k_sems.at[s])
            vc = pltpu.make_async_copy(
                v_hbm.at[0, :, h0 + h, :], vstage.at[s], v_sems.at[s])
            kc.start()
            vc.start()
            kv_cp[h] = (kc, vc)

        start_kv(0)
        start_kv(1)

        xb[...] = x_ref[...].astype(BF16)
        wqb[...] = wq_ref[...].astype(BF16)
        wob[...] = wo_ref[...].astype(BF16)

        bsem = pltpu.get_barrier_semaphore()
        pl.semaphore_signal(bsem, inc=1, device_id=(left,),
                            device_id_type=pl.DeviceIdType.MESH)
        pl.semaphore_signal(bsem, inc=1, device_id=(right,),
                            device_id_type=pl.DeviceIdType.MESH)
        pl.semaphore_wait(bsem, 2)

        def ag_rdma(hop):
            src = xb if hop == 0 else comm.at[hop - 1]
            return pltpu.make_async_remote_copy(
                src_ref=src, dst_ref=comm.at[hop],
                send_sem=ag_ssem.at[hop], recv_sem=ag_rsem.at[hop],
                device_id=(right,), device_id_type=pl.DeviceIdType.MESH)

        def rs_rdma(t):
            return pltpu.make_async_remote_copy(
                src_ref=rs_send.at[t], dst_ref=rs_recv.at[t],
                send_sem=rs_ssem.at[t], recv_sem=rs_rsem.at[t],
                device_id=(right,), device_id_type=pl.DeviceIdType.MESH)

        ag = [ag_rdma(hop) for hop in range(N_DEV - 1)]
        rs = [rs_rdma(t) for t in range(N_DEV - 1)]

        def attn_block(b, first=False):
            ws = pl.multiple_of(jnp.maximum(b * SQ - 128, 0), 128)
            q0 = b * SQ
            for h in range(H_LOC):
                if first:
                    kc, vc = kv_cp[h]
                    kc.wait()
                    vc.wait()
                    kb[h] = kstage[h % 2].astype(BF16)
                    vb[h] = vstage[h % 2].astype(BF16)
                    if h + 2 < H_LOC:
                        start_kv(h + 2)
                qb = q_ref[pl.ds(q0, SQ), h * DH:(h + 1) * DH]
                kw = kb[h, pl.ds(ws, WIN), :]
                sw = _dot_nt(qb, kw) * SCALE
                qi = lax.broadcasted_iota(jnp.int32, (SQ, WIN), 0) + q0
                ki = lax.broadcasted_iota(jnp.int32, (SQ, WIN), 1) + ws
                mask = (jnp.abs(qi - ki) <= 128) | ((ki < G) & (ws == 0))
                sw = jnp.where(mask, sw, NEG)
                kp = kb[h, 0:DH, :]
                sp = _dot_nt(qb, kp) * SCALE
                kip = lax.broadcasted_iota(jnp.int32, (SQ, DH), 1)
                sp = jnp.where((kip < G) & (ws > 0), sp, NEG)
                m = jnp.maximum(jnp.max(sw, axis=1, keepdims=True),
                                jnp.max(sp, axis=1, keepdims=True))
                ew = jnp.exp(sw - m)
                ep = jnp.exp(sp - m)
                den = (jnp.sum(ew, axis=1, keepdims=True)
                       + jnp.sum(ep, axis=1, keepdims=True))
                ctx = (_dot((ew / den).astype(BF16), vb[h, pl.ds(ws, WIN), :])
                       + _dot((ep / den).astype(BF16), vb[h, 0:DH, :]))
                rowi = lax.broadcasted_iota(jnp.int32, (SQ, DH), 0) + q0
                ctx = jnp.where(rowi < G, 0.0, ctx)
                ctxbuf[:, h * DH:(h + 1) * DH] = ctx.astype(BF16)
            partial[pl.ds(q0, SQ), :] = _dot(ctxbuf[...], wob[...])

            @pl.when(b == 0)
            def _():
                for h in range(H_LOC):
                    qg = q_ref[0:G, h * DH:(h + 1) * DH]
                    sg = _dot_nt(qg, kb[h]) * SCALE
                    mg = jnp.max(sg, axis=1, keepdims=True)
                    eg = jnp.exp(sg - mg)
                    wg = eg / jnp.sum(eg, axis=1, keepdims=True)
                    ctx_g = _dot(wg.astype(BF16), vb[h])
                    g = _dot(ctx_g.astype(BF16),
                             wob[h * DH:(h + 1) * DH, :])
                    gsum = g if h == 0 else gsum + g
                partial[0:G, :] = partial[0:G, :] + gsum

        ag[0].start()
        q_ref[pl.ds(me * SQ, SQ), :] = _dot(xb[...], wqb[...]).astype(BF16)
        attn_block(me, first=True)

        for p in range(1, N_DEV):
            ag[p - 1].wait_recv()
            if p < N_DEV - 1:
                ag[p].start()
            b = (me + N_DEV - p) % N_DEV
            q_ref[pl.ds(b * SQ, SQ), :] = _dot(
                comm[p - 1], wqb[...]).astype(BF16)
            if p == 2:
                blk0 = (me + 3) % N_DEV
                rs_send[0] = partial[pl.ds(blk0 * SQ, SQ), :].astype(BF16)
                rs[0].start()
            attn_block(b)
            if p == 2:
                rs[0].wait_recv()
                blk1 = (me + 2) % N_DEV
                rs_send[1] = (rs_recv[0] + partial[pl.ds(blk1 * SQ, SQ), :]
                              ).astype(BF16)
                rs[1].start()
            if p == 3:
                rs[1].wait_recv()
                blk2 = (me + 1) % N_DEV
                rs_send[2] = (rs_recv[1] + partial[pl.ds(blk2 * SQ, SQ), :]
                              ).astype(BF16)
                rs[2].start()

        rs[2].wait_recv()
        out_ref[...] = rs_recv[2] + partial[pl.ds(me * SQ, SQ), :]

        for r in ag + rs:
            r.wait_send()

    out = pl.pallas_call(
        body,
        out_shape=jax.ShapeDtypeStruct((SQ, D_MODEL), F32),
        in_specs=[
            pl.BlockSpec(memory_space=pltpu.MemorySpace.VMEM),
            pl.BlockSpec(memory_space=pltpu.MemorySpace.VMEM),
            pl.BlockSpec(memory_space=pl.ANY),
            pl.BlockSpec(memory_space=pl.ANY),
            pl.BlockSpec(memory_space=pltpu.MemorySpace.VMEM),
        ],
        out_specs=pl.BlockSpec(memory_space=pltpu.MemorySpace.VMEM),
        scratch_shapes=[
            pltpu.VMEM((SQ, D_MODEL), BF16),
            pltpu.VMEM((N_DEV - 1, SQ, D_MODEL), BF16),
            pltpu.VMEM((N_DEV * SQ, D_MODEL), BF16),
            pltpu.VMEM((2, SKV, DH), F32),
            pltpu.VMEM((2, SKV, DH), F32),
            pltpu.VMEM((H_LOC, SKV, DH), BF16),
            pltpu.VMEM((H_LOC, SKV, DH), BF16),
            pltpu.VMEM((D_MODEL, D_MODEL), BF16),
            pltpu.VMEM((D_MODEL, D_MODEL), BF16),
            pltpu.VMEM((SQ, D_MODEL), BF16),
            pltpu.VMEM((N_DEV * SQ, D_MODEL), F32),
            pltpu.VMEM((N_DEV - 1, SQ, D_MODEL), BF16),
            pltpu.VMEM((N_DEV - 1, SQ, D_MODEL), BF16),
            pltpu.SemaphoreType.DMA((N_DEV - 1,)),
            pltpu.SemaphoreType.DMA((N_DEV - 1,)),
            pltpu.SemaphoreType.DMA((2,)),
            pltpu.SemaphoreType.DMA((2,)),
            pltpu.SemaphoreType.DMA((N_DEV - 1,)),
            pltpu.SemaphoreType.DMA((N_DEV - 1,)),
        ],
        compiler_params=pltpu.CompilerParams(
            collective_id=0, vmem_limit_bytes=100 * 1024 * 1024),
    )(x2, Wq, K_ext, V_ext, Wo)
    return out.reshape(1, SQ, D_MODEL)


# device time: 65332 ns/iter; 3.2371x vs baseline; 1.0244x over previous
import jax
import jax.numpy as jnp
from jax import lax
from jax.experimental import pallas as pl
from jax.experimental.pallas import tpu as pltpu

N_DEV = 4
SQ = 256
D_MODEL = 1024
SKV = 4096
H_TOT = 32
H_LOC = 8
DH = 128
SCALE = 0.08838834764831843
WIN = 512
G = 32
NEG = -1e9
F32 = jnp.float32
BF16 = jnp.bfloat16


def _dot_nt(a, b):
    return lax.dot_general(a, b, (((1,), (1,)), ((), ())),
                           preferred_element_type=F32)


def _dot(a, b):
    return jnp.dot(a, b, preferred_element_type=F32)


def kernel(x, Wq, K_ext, V_ext, Wo):
    x2 = x.reshape(SQ, D_MODEL)

    def body(x_ref, wq_ref, k_hbm, v_hbm, wo_ref, out_ref,
             xb, comm, q_ref, kstage, vstage, kb, vb, wqb, wob, ctxbuf,
             partial, rs_recv, rs_send,
             ag_ssem, ag_rsem, k_sems, v_sems, rs_ssem, rs_rsem):
        me = lax.axis_index("i")
        right = (me + 1) % N_DEV
        left = (me + 3) % N_DEV
        h0 = me * H_LOC

        kv_cp = {}

        def start_kv(h):
            s = h % 2
            kc = pltpu.make_async_copy(
                k_hbm.at[0, :, h0 + h, :], kstage.at[s], k_sems.at[s])
            vc = pltpu.make_async_copy(
                v_hbm.at[0, :, h0 + h, :], vstage.at[s], v_sems.at[s])
            kc.start()
            vc.start()
            kv_cp[h] = (kc, vc)

        start_kv(0)
        start_kv(1)

        xb[...] = x_ref[...].astype(BF16)
        wqb[...] = wq_ref[...].astype(BF16)
        wob[...] = wo_ref[...].astype(BF16)

        bsem = pltpu.get_barrier_semaphore()
        pl.semaphore_signal(bsem, inc=1, device_id=(left,),
                            device_id_type=pl.DeviceIdType.MESH)
        pl.semaphore_signal(bsem, inc=1, device_id=(right,),
                            device_id_type=pl.DeviceIdType.MESH)
        pl.semaphore_wait(bsem, 2)

        def ag_rdma(hop):
            src = xb if hop == 0 else comm.at[hop - 1]
            return pltpu.make_async_remote_copy(
                src_ref=src, dst_ref=comm.at[hop],
                send_sem=ag_ssem.at[hop], recv_sem=ag_rsem.at[hop],
                device_id=(right,), device_id_type=pl.DeviceIdType.MESH)

        def rs_rdma(t):
            return pltpu.make_async_remote_copy(
                src_ref=rs_send.at[t], dst_ref=rs_recv.at[t],
                send_sem=rs_ssem.at[t], recv_sem=rs_rsem.at[t],
                device_id=(right,), device_id_type=pl.DeviceIdType.MESH)

        ag = [ag_rdma(hop) for hop in range(N_DEV - 1)]
        rs = [rs_rdma(t) for t in range(N_DEV - 1)]

        def attn_block(b, first=False):
            ws = pl.multiple_of(jnp.maximum(b * SQ - 128, 0), 128)
            q0 = b * SQ
            qi = lax.broadcasted_iota(jnp.int32, (SQ, WIN), 0) + q0
            ki = lax.broadcasted_iota(jnp.int32, (SQ, WIN), 1) + ws
            wmask = (jnp.abs(qi - ki) <= 128) | ((ki < G) & (ws == 0))
            wbias = jnp.where(wmask, 0.0, NEG)
            kip = lax.broadcasted_iota(jnp.int32, (SQ, DH), 1)
            pbias = jnp.where((kip < G) & (ws > 0), 0.0, NEG)
            rowi = lax.broadcasted_iota(jnp.int32, (SQ, 1), 0) + q0
            rowkeep = (rowi >= G).astype(F32)
            for h in range(H_LOC):
                if first:
                    kc, vc = kv_cp[h]
                    kc.wait()
                    vc.wait()
                    kb[h] = kstage[h % 2].astype(BF16)
                    vb[h] = vstage[h % 2].astype(BF16)
                    if h + 2 < H_LOC:
                        start_kv(h + 2)
                qb = q_ref[pl.ds(q0, SQ), h * DH:(h + 1) * DH]
                kw = kb[h, pl.ds(ws, WIN), :]
                sw = _dot_nt(qb, kw) + wbias
                kp = kb[h, 0:DH, :]
                sp = _dot_nt(qb, kp) + pbias
                m = jnp.maximum(jnp.max(sw, axis=1, keepdims=True),
                                jnp.max(sp, axis=1, keepdims=True))
                ew = jnp.exp(sw - m)
                ep = jnp.exp(sp - m)
                den = (jnp.sum(ew, axis=1, keepdims=True)
                       + jnp.sum(ep, axis=1, keepdims=True))
                ctx = (_dot(ew.astype(BF16), vb[h, pl.ds(ws, WIN), :])
                       + _dot(ep.astype(BF16), vb[h, 0:DH, :]))
                ctx = ctx * (rowkeep / den)
                ctxbuf[:, h * DH:(h + 1) * DH] = ctx.astype(BF16)
            partial[pl.ds(q0, SQ), :] = _dot(ctxbuf[...], wob[...])

            @pl.when(b == 0)
            def _():
                for h in range(H_LOC):
                    qg = q_ref[0:G, h * DH:(h + 1) * DH]
                    sg = _dot_nt(qg, kb[h])
                    mg = jnp.max(sg, axis=1, keepdims=True)
                    eg = jnp.exp(sg - mg)
                    ctx_g = (_dot(eg.astype(BF16), vb[h])
                             / jnp.sum(eg, axis=1, keepdims=True))
                    g = _dot(ctx_g.astype(BF16),
                             wob[h * DH:(h + 1) * DH, :])
                    gsum = g if h == 0 else gsum + g
                partial[0:G, :] = partial[0:G, :] + gsum

        ag[0].start()
        q_ref[pl.ds(me * SQ, SQ), :] = (
            _dot(xb[...], wqb[...]) * SCALE).astype(BF16)
        attn_block(me, first=True)

        for p in range(1, N_DEV):
            ag[p - 1].wait_recv()
            if p < N_DEV - 1:
                ag[p].start()
            b = (me + N_DEV - p) % N_DEV
            q_ref[pl.ds(b * SQ, SQ), :] = (
                _dot(comm[p - 1], wqb[...]) * SCALE).astype(BF16)
            if p == 2:
                blk0 = (me + 3) % N_DEV
                rs_send[0] = partial[pl.ds(blk0 * SQ, SQ), :].astype(BF16)
                rs[0].start()
            attn_block(b)
            if p == 2:
                rs[0].wait_recv()
                blk1 = (me + 2) % N_DEV
                rs_send[1] = (rs_recv[0] + partial[pl.ds(blk1 * SQ, SQ), :]
                              ).astype(BF16)
                rs[1].start()
            if p == 3:
                rs[1].wait_recv()
                blk2 = (me + 1) % N_DEV
                rs_send[2] = (rs_recv[1] + partial[pl.ds(blk2 * SQ, SQ), :]
                              ).astype(BF16)
                rs[2].start()

        rs[2].wait_recv()
        out_ref[...] = rs_recv[2] + partial[pl.ds(me * SQ, SQ), :]

        for r in ag + rs:
            r.wait_send()

    out = pl.pallas_call(
        body,
        out_shape=jax.ShapeDtypeStruct((SQ, D_MODEL), F32),
        in_specs=[
            pl.BlockSpec(memory_space=pltpu.MemorySpace.VMEM),
            pl.BlockSpec(memory_space=pltpu.MemorySpace.VMEM),
            pl.BlockSpec(memory_space=pl.ANY),
            pl.BlockSpec(memory_space=pl.ANY),
            pl.BlockSpec(memory_space=pltpu.MemorySpace.VMEM),
        ],
        out_specs=pl.BlockSpec(memory_space=pltpu.MemorySpace.VMEM),
        scratch_shapes=[
            pltpu.VMEM((SQ, D_MODEL), BF16),
            pltpu.VMEM((N_DEV - 1, SQ, D_MODEL), BF16),
            pltpu.VMEM((N_DEV * SQ, D_MODEL), BF16),
            pltpu.VMEM((2, SKV, DH), F32),
            pltpu.VMEM((2, SKV, DH), F32),
            pltpu.VMEM((H_LOC, SKV, DH), BF16),
            pltpu.VMEM((H_LOC, SKV, DH), BF16),
            pltpu.VMEM((D_MODEL, D_MODEL), BF16),
            pltpu.VMEM((D_MODEL, D_MODEL), BF16),
            pltpu.VMEM((SQ, D_MODEL), BF16),
            pltpu.VMEM((N_DEV * SQ, D_MODEL), F32),
            pltpu.VMEM((N_DEV - 1, SQ, D_MODEL), BF16),
            pltpu.VMEM((N_DEV - 1, SQ, D_MODEL), BF16),
            pltpu.SemaphoreType.DMA((N_DEV - 1,)),
            pltpu.SemaphoreType.DMA((N_DEV - 1,)),
            pltpu.SemaphoreType.DMA((2,)),
            pltpu.SemaphoreType.DMA((2,)),
            pltpu.SemaphoreType.DMA((N_DEV - 1,)),
            pltpu.SemaphoreType.DMA((N_DEV - 1,)),
        ],
        compiler_params=pltpu.CompilerParams(
            collective_id=0, vmem_limit_bytes=100 * 1024 * 1024),
    )(x2, Wq, K_ext, V_ext, Wo)
    return out.reshape(1, SQ, D_MODEL)


# device time: 51386 ns/iter; 4.1156x vs baseline; 1.2714x over previous
import jax
import jax.numpy as jnp
from jax import lax
from jax.experimental import pallas as pl
from jax.experimental.pallas import tpu as pltpu

N_DEV = 4
SQ = 256
D_MODEL = 1024
SKV = 4096
H_TOT = 32
H_LOC = 8
DH = 128
SCALE = 0.08838834764831843
WIN = 512
G = 32
NEG = -1e9
F32 = jnp.float32
BF16 = jnp.bfloat16

DO_COMM = True
DO_ATTN = True


def _dot_nt(a, b):
    return lax.dot_general(a, b, (((1,), (1,)), ((), ())),
                           preferred_element_type=F32)


def _dot(a, b):
    return jnp.dot(a, b, preferred_element_type=F32)


def kernel(x, Wq, K_ext, V_ext, Wo):
    x2 = x.reshape(SQ, D_MODEL)

    def body(x_ref, wq_ref, k_hbm, v_hbm, wo_ref, out_ref,
             xb, comm, lcomm, q_ref, kstage, vstage, kb, vb, wqb, wob,
             ctxbuf, partial, rs_lbuf, rs_rbuf, rs_dbuf,
             rsl_recv, rsr_recv, rsd_recv,
             ag_ssem, ag_rsem, lag_ssem, lag_rsem,
             k_sems, v_sems, rs_ssem, rs_rsem):
        me = lax.axis_index("i")
        right = (me + 1) % N_DEV
        left = (me + 3) % N_DEV
        h0 = me * H_LOC

        kv_cp = {}

        def start_kv(h):
            sl = h % 2
            kc = pltpu.make_async_copy(
                k_hbm.at[0, :, h0 + h, :], kstage.at[sl], k_sems.at[sl])
            vc = pltpu.make_async_copy(
                v_hbm.at[0, :, h0 + h, :], vstage.at[sl], v_sems.at[sl])
            kc.start()
            vc.start()
            kv_cp[h] = (kc, vc)

        if DO_ATTN:
            start_kv(0)
            start_kv(1)

        xb[...] = x_ref[...].astype(BF16)
        wqb[...] = wq_ref[...].astype(BF16)
        wob[...] = wo_ref[...].astype(BF16)

        if DO_COMM:
            bsem = pltpu.get_barrier_semaphore()
            pl.semaphore_signal(bsem, inc=1, device_id=(left,),
                                device_id_type=pl.DeviceIdType.MESH)
            pl.semaphore_signal(bsem, inc=1, device_id=(right,),
                                device_id_type=pl.DeviceIdType.MESH)
            pl.semaphore_wait(bsem, 2)

        ag0r = pltpu.make_async_remote_copy(
            src_ref=xb, dst_ref=comm.at[0],
            send_sem=ag_ssem.at[0], recv_sem=ag_rsem.at[0],
            device_id=(right,), device_id_type=pl.DeviceIdType.MESH)
        ag1 = pltpu.make_async_remote_copy(
            src_ref=comm.at[0], dst_ref=comm.at[1],
            send_sem=ag_ssem.at[1], recv_sem=ag_rsem.at[1],
            device_id=(right,), device_id_type=pl.DeviceIdType.MESH)
        ag0l = pltpu.make_async_remote_copy(
            src_ref=xb, dst_ref=lcomm,
            send_sem=lag_ssem, recv_sem=lag_rsem,
            device_id=(left,), device_id_type=pl.DeviceIdType.MESH)
        rs_l = pltpu.make_async_remote_copy(
            src_ref=rs_lbuf, dst_ref=rsl_recv,
            send_sem=rs_ssem.at[0], recv_sem=rs_rsem.at[0],
            device_id=(left,), device_id_type=pl.DeviceIdType.MESH)
        rs_r = pltpu.make_async_remote_copy(
            src_ref=rs_rbuf, dst_ref=rsr_recv,
            send_sem=rs_ssem.at[1], recv_sem=rs_rsem.at[1],
            device_id=(right,), device_id_type=pl.DeviceIdType.MESH)
        rs_d = pltpu.make_async_remote_copy(
            src_ref=rs_dbuf, dst_ref=rsd_recv,
            send_sem=rs_ssem.at[2], recv_sem=rs_rsem.at[2],
            device_id=((me + 2) % N_DEV,),
            device_id_type=pl.DeviceIdType.MESH)

        def attn_block(b):
            ws = pl.multiple_of(jnp.maximum(b * SQ - 128, 0), 128)
            q0 = b * SQ
            qi = lax.broadcasted_iota(jnp.int32, (SQ, WIN), 0) + q0
            ki = lax.broadcasted_iota(jnp.int32, (SQ, WIN), 1) + ws
            wmask = (jnp.abs(qi - ki) <= 128) | ((ki < G) & (ws == 0))
            wbias = jnp.where(wmask, 0.0, NEG)
            kip = lax.broadcasted_iota(jnp.int32, (SQ, DH), 1)
            pbias = jnp.where((kip < G) & (ws > 0), 0.0, NEG)
            rowi = lax.broadcasted_iota(jnp.int32, (SQ, 1), 0) + q0
            rowkeep = (rowi >= G).astype(F32)
            for h in range(H_LOC if DO_ATTN else 0):
                qb = q_ref[pl.ds(q0, SQ), h * DH:(h + 1) * DH]
                kw = kb[h, pl.ds(ws, WIN), :]
                sw = _dot_nt(qb, kw) + wbias
                kp = kb[h, 0:DH, :]
                sp = _dot_nt(qb, kp) + pbias
                m = jnp.maximum(jnp.max(sw, axis=1, keepdims=True),
                                jnp.max(sp, axis=1, keepdims=True))
                ew = jnp.exp(sw - m)
                ep = jnp.exp(sp - m)
                den = (jnp.sum(ew, axis=1, keepdims=True)
                       + jnp.sum(ep, axis=1, keepdims=True))
                ctx = (_dot(ew.astype(BF16), vb[h, pl.ds(ws, WIN), :])
                       + _dot(ep.astype(BF16), vb[h, 0:DH, :]))
                ctx = ctx * (rowkeep / den)
                ctxbuf[:, h * DH:(h + 1) * DH] = ctx.astype(BF16)
            partial[pl.ds(q0, SQ), :] = _dot(ctxbuf[...], wob[...])

            @pl.when(jnp.logical_and(b == 0, DO_ATTN))
            def _():
                for h in range(H_LOC):
                    qg = q_ref[0:G, h * DH:(h + 1) * DH]
                    sg = _dot_nt(qg, kb[h])
                    mg = jnp.max(sg, axis=1, keepdims=True)
                    eg = jnp.exp(sg - mg)
                    ctx_g = (_dot(eg.astype(BF16), vb[h])
                             / jnp.sum(eg, axis=1, keepdims=True))
                    g = _dot(ctx_g.astype(BF16),
                             wob[h * DH:(h + 1) * DH, :])
                    gsum = g if h == 0 else gsum + g
                partial[0:G, :] = partial[0:G, :] + gsum

        if DO_COMM:
            ag0r.start()
            ag0l.start()
        q_ref[pl.ds(me * SQ, SQ), :] = (
            _dot(xb[...], wqb[...]) * SCALE).astype(BF16)
        for h in range(H_LOC if DO_ATTN else 0):
            kc, vc = kv_cp[h]
            kc.wait()
            vc.wait()
            kb[h] = kstage[h % 2].astype(BF16)
            vb[h] = vstage[h % 2].astype(BF16)
            if h + 2 < H_LOC:
                start_kv(h + 2)

        if DO_COMM:
            ag0r.wait_recv()
            ag1.start()
        b1 = (me + 3) % N_DEV
        q_ref[pl.ds(b1 * SQ, SQ), :] = (
            _dot(comm[0], wqb[...]) * SCALE).astype(BF16)
        attn_block(b1)
        rs_lbuf[...] = partial[pl.ds(b1 * SQ, SQ), :].astype(BF16)
        if DO_COMM:
            rs_l.start()

        if DO_COMM:
            ag0l.wait_recv()
        b2 = (me + 1) % N_DEV
        q_ref[pl.ds(b2 * SQ, SQ), :] = (
            _dot(lcomm[...], wqb[...]) * SCALE).astype(BF16)
        attn_block(b2)
        rs_rbuf[...] = partial[pl.ds(b2 * SQ, SQ), :].astype(BF16)
        if DO_COMM:
            rs_r.start()

        if DO_COMM:
            ag1.wait_recv()
        b3 = (me + 2) % N_DEV
        q_ref[pl.ds(b3 * SQ, SQ), :] = (
            _dot(comm[1], wqb[...]) * SCALE).astype(BF16)
        attn_block(b3)
        rs_dbuf[...] = partial[pl.ds(b3 * SQ, SQ), :].astype(BF16)
        if DO_COMM:
            rs_d.start()

        attn_block(me)
        if DO_COMM:
            rs_l.wait_recv()
            rs_r.wait_recv()
            rs_d.wait_recv()
        out_ref[...] = ((partial[pl.ds(me * SQ, SQ), :]
                         + rsl_recv[...].astype(F32))
                        + (rsr_recv[...].astype(F32)
                           + rsd_recv[...].astype(F32)))

        if DO_COMM:
            for r in (ag0r, ag0l, ag1, rs_l, rs_r, rs_d):
                r.wait_send()

    out = pl.pallas_call(
        body,
        out_shape=jax.ShapeDtypeStruct((SQ, D_MODEL), F32),
        in_specs=[
            pl.BlockSpec(memory_space=pltpu.MemorySpace.VMEM),
            pl.BlockSpec(memory_space=pltpu.MemorySpace.VMEM),
            pl.BlockSpec(memory_space=pl.ANY),
            pl.BlockSpec(memory_space=pl.ANY),
            pl.BlockSpec(memory_space=pltpu.MemorySpace.VMEM),
        ],
        out_specs=pl.BlockSpec(memory_space=pltpu.MemorySpace.VMEM),
        scratch_shapes=[
            pltpu.VMEM((SQ, D_MODEL), BF16),
            pltpu.VMEM((2, SQ, D_MODEL), BF16),
            pltpu.VMEM((SQ, D_MODEL), BF16),
            pltpu.VMEM((N_DEV * SQ, D_MODEL), BF16),
            pltpu.VMEM((2, SKV, DH), F32),
            pltpu.VMEM((2, SKV, DH), F32),
            pltpu.VMEM((H_LOC, SKV, DH), BF16),
            pltpu.VMEM((H_LOC, SKV, DH), BF16),
            pltpu.VMEM((D_MODEL, D_MODEL), BF16),
            pltpu.VMEM((D_MODEL, D_MODEL), BF16),
            pltpu.VMEM((SQ, D_MODEL), BF16),
            pltpu.VMEM((N_DEV * SQ, D_MODEL), F32),
            pltpu.VMEM((SQ, D_MODEL), BF16),
            pltpu.VMEM((SQ, D_MODEL), BF16),
            pltpu.VMEM((SQ, D_MODEL), BF16),
            pltpu.VMEM((SQ, D_MODEL), BF16),
            pltpu.VMEM((SQ, D_MODEL), BF16),
            pltpu.VMEM((SQ, D_MODEL), BF16),
            pltpu.SemaphoreType.DMA((2,)),
            pltpu.SemaphoreType.DMA((2,)),
            pltpu.SemaphoreType.DMA,
            pltpu.SemaphoreType.DMA,
            pltpu.SemaphoreType.DMA((2,)),
            pltpu.SemaphoreType.DMA((2,)),
            pltpu.SemaphoreType.DMA((3,)),
            pltpu.SemaphoreType.DMA((3,)),
        ],
        compiler_params=pltpu.CompilerParams(
            collective_id=0 if DO_COMM else None,
            vmem_limit_bytes=110 * 1024 * 1024),
    )(x2, Wq, K_ext, V_ext, Wo)
    return out.reshape(1, SQ, D_MODEL)
